# Initial kernel scaffold; baseline (speedup 1.0000x reference)
import functools

import jax
import jax.numpy as jnp
from jax import lax
from jax.experimental import pallas as pl
from jax.experimental.pallas import tpu as pltpu
from jax.experimental.pallas import tpu_sc as plsc

B = 16384
D = 64
N = 5
NC = 2
NS = 16
NW = NC * NS
BPW = B // NW
CH = 128
NCH = BPW // CH
L = 16


def _sc_body(pos_w_hbm, pos_v_hbm, neg_hbm, w_hbm, v_hbm, out_hbm,
             widx, vidx, nidx, wrows, vrows, nrows, outv, sem):
    cid = lax.axis_index("c")
    sid = lax.axis_index("s")
    wid = sid * NC + cid
    cbase = wid * NCH
    nbase = wid * (NCH * N)
    obase = wid * BPW

    pltpu.sync_copy(pos_w_hbm.at[pl.ds(cbase, NCH)], widx)
    pltpu.sync_copy(pos_v_hbm.at[pl.ds(cbase, NCH)], vidx)
    pltpu.sync_copy(neg_hbm.at[pl.ds(nbase, NCH * N)], nidx)

    iota = lax.iota(jnp.int32, L)

    for ch in range(NCH):
        pltpu.async_copy(w_hbm.at[widx.at[ch]], wrows, sem).wait()
        pltpu.async_copy(v_hbm.at[vidx.at[ch]], vrows, sem).wait()
        for k in range(N):
            pltpu.async_copy(v_hbm.at[nidx.at[ch * N + k]], nrows.at[k], sem).wait()

        for g in range(CH // L):
            row = g * L + iota

            def d_body(d, accs, row=row):
                dcol = jnp.full((L,), d, jnp.int32)
                vv = plsc.load_gather(vrows, [row, dcol])
                wv = plsc.load_gather(wrows, [row, dcol])
                out = [accs[0] + wv * vv]
                for n in range(N):
                    nv = plsc.load_gather(nrows.at[n], [row, dcol])
                    out.append(accs[1 + n] + nv * vv)
                return tuple(out)

            z = jnp.zeros((L,), jnp.float32)
            accs = lax.fori_loop(0, D, d_body, (z,) * (1 + N))
            off = ch * CH + g * L
            outv[0, pl.ds(off, L)] = accs[0]
            for n in range(N):
                outv[1 + n, pl.ds(off, L)] = -accs[1 + n]

    for j in range(1 + N):
        pltpu.sync_copy(outv.at[j], out_hbm.at[pl.ds(j * B + obase, BPW)])


def _sc_dots(pos_w2, pos_v2, neg2, w_emb, v_emb):
    mesh = plsc.VectorSubcoreMesh(core_axis_name="c", subcore_axis_name="s",
                                  num_cores=NC, num_subcores=NS)
    f = pl.kernel(
        _sc_body,
        out_type=jax.ShapeDtypeStruct(((1 + N) * B,), jnp.float32),
        mesh=mesh,
        scratch_types=[
            pltpu.VMEM((NCH, CH), jnp.int32),
            pltpu.VMEM((NCH, CH), jnp.int32),
            pltpu.VMEM((NCH * N, CH), jnp.int32),
            pltpu.VMEM((CH, D), jnp.float32),
            pltpu.VMEM((CH, D), jnp.float32),
            pltpu.VMEM((N, CH, D), jnp.float32),
            pltpu.VMEM((1 + N, BPW), jnp.float32),
            pltpu.SemaphoreType.DMA,
        ],
    )
    return f(pos_w2, pos_v2, neg2, w_emb, v_emb)


def _tc_body(x_ref, o_ref):
    x = x_ref[...]
    y = jnp.minimum(x, 0.0) - jnp.log1p(jnp.exp(-jnp.abs(x)))
    o_ref[0, 0] = -jnp.sum(y)


def kernel(pos_w, pos_v, neg_v, w_emb, v_emb):
    pos_w2 = pos_w.astype(jnp.int32).reshape(B // CH, CH)
    pos_v2 = pos_v.astype(jnp.int32).reshape(B // CH, CH)
    neg2 = neg_v.astype(jnp.int32).reshape(B * N // CH, CH)
    raw = _sc_dots(pos_w2, pos_v2, neg2, w_emb, v_emb)
    loss = pl.pallas_call(
        _tc_body,
        out_shape=jax.ShapeDtypeStruct((1, 1), jnp.float32),
    )(raw.reshape((1 + N) * B // 128, 128))
    return loss[0, 0]


# speedup vs baseline: 1.5504x; 1.5504x over previous
import jax
import jax.numpy as jnp
from jax import lax
from jax.experimental import pallas as pl
from jax.experimental.pallas import tpu as pltpu
from jax.experimental.pallas import tpu_sc as plsc

B = 16384
D = 64
N = 5
NC = 2
NS = 16
NW = NC * NS
BPW = B // NW
CH = 128
NCH = BPW // CH
L = 16


def _sc_body(pos_w_hbm, pos_v_hbm, neg_hbm, w_hbm, v_hbm, out_hbm,
             widx, vidx, nidx2, nflat, wrows, vrows, nrows, outv, sem):
    cid = lax.axis_index("c")
    sid = lax.axis_index("s")
    wid = sid * NC + cid
    obase = wid * BPW

    pltpu.sync_copy(pos_w_hbm.at[pl.ds(obase, BPW)], widx)
    pltpu.sync_copy(pos_v_hbm.at[pl.ds(obase, BPW)], vidx)
    pltpu.sync_copy(neg_hbm.at[pl.ds(obase, BPW)], nidx2)

    iota = lax.iota(jnp.int32, L)

    def repack(i, _):
        j = i * L + iota
        r = j // N
        c = j - r * N
        nflat[pl.ds(i * L, L)] = plsc.load_gather(nidx2, [r, c])
        return 0
    lax.fori_loop(0, BPW * N // L, repack, 0)

    for ch in range(NCH):
        pltpu.async_copy(w_hbm.at[widx.at[pl.ds(ch * CH, CH)]], wrows, sem).wait()
        pltpu.async_copy(v_hbm.at[vidx.at[pl.ds(ch * CH, CH)]], vrows, sem).wait()
        for k in range(N):
            pltpu.async_copy(
                v_hbm.at[nflat.at[pl.ds((ch * N + k) * CH, CH)]],
                nrows.at[pl.ds(k * CH, CH)], sem).wait()

        for g in range(CH // L):
            row = g * L + iota
            rowx5 = row * N

            def d_body(d, accs, row=row, rowx5=rowx5):
                dcol = jnp.full((L,), d, jnp.int32)
                vv = plsc.load_gather(vrows, [row, dcol])
                wv = plsc.load_gather(wrows, [row, dcol])
                out = [accs[0] + wv * vv]
                for n in range(N):
                    nv = plsc.load_gather(nrows, [rowx5 + n, dcol])
                    out.append(accs[1 + n] + nv * vv)
                return tuple(out)

            z = jnp.zeros((L,), jnp.float32)
            accs = lax.fori_loop(0, D, d_body, (z,) * (1 + N))
            off = ch * CH + g * L
            outv[0, pl.ds(off, L)] = accs[0]
            for n in range(N):
                outv[1 + n, pl.ds(off, L)] = -accs[1 + n]

    for j in range(1 + N):
        pltpu.sync_copy(outv.at[j], out_hbm.at[pl.ds(j * B + obase, BPW)])


def _sc_dots(pos_w, pos_v, neg_v, w_emb, v_emb):
    mesh = plsc.VectorSubcoreMesh(core_axis_name="c", subcore_axis_name="s",
                                  num_cores=NC, num_subcores=NS)
    f = pl.kernel(
        _sc_body,
        out_type=jax.ShapeDtypeStruct(((1 + N) * B,), jnp.float32),
        mesh=mesh,
        compiler_params=pltpu.CompilerParams(needs_layout_passes=False,
                                             use_tc_tiling_on_sc=False),
        scratch_types=[
            pltpu.VMEM((BPW,), jnp.int32),
            pltpu.VMEM((BPW,), jnp.int32),
            pltpu.VMEM((BPW, N), jnp.int32),
            pltpu.VMEM((BPW * N,), jnp.int32),
            pltpu.VMEM((CH, D), jnp.float32),
            pltpu.VMEM((CH, D), jnp.float32),
            pltpu.VMEM((N * CH, D), jnp.float32),
            pltpu.VMEM((1 + N, BPW), jnp.float32),
            pltpu.SemaphoreType.DMA,
        ],
    )
    return f(pos_w, pos_v, neg_v, w_emb, v_emb)


def _tc_body(x_ref, o_ref):
    x = x_ref[...]
    y = jnp.minimum(x, 0.0) - jnp.log1p(jnp.exp(-jnp.abs(x)))
    o_ref[...] = -jnp.sum(y, keepdims=True)


def kernel(pos_w, pos_v, neg_v, w_emb, v_emb):
    raw = _sc_dots(pos_w.astype(jnp.int32), pos_v.astype(jnp.int32),
                   neg_v.astype(jnp.int32), w_emb, v_emb)
    loss = pl.pallas_call(
        _tc_body,
        out_shape=jax.ShapeDtypeStruct((1, 1), jnp.float32),
    )(raw.reshape((1 + N) * B // 128, 128))
    return loss[0, 0]


# speedup vs baseline: 1.5819x; 1.0203x over previous
import jax
import jax.numpy as jnp
from jax import lax
from jax.experimental import pallas as pl
from jax.experimental.pallas import tpu as pltpu
from jax.experimental.pallas import tpu_sc as plsc

B = 16384
D = 64
N = 5
NC = 2
NS = 16
NW = NC * NS
BPW = B // NW
CH = 128
NCH = BPW // CH
L = 16


def _sc_body(pos_w_hbm, pos_v_hbm, neg_hbm, w_hbm, v_hbm, out_hbm,
             widx, vidx, nidx2, nflat, wrows, vrows, nrows, outv, sem):
    cid = lax.axis_index("c")
    sid = lax.axis_index("s")
    wid = sid * NC + cid
    obase = wid * BPW

    pltpu.sync_copy(pos_w_hbm.at[pl.ds(obase, BPW)], widx)
    pltpu.sync_copy(pos_v_hbm.at[pl.ds(obase, BPW)], vidx)
    pltpu.sync_copy(neg_hbm.at[pl.ds(obase, BPW)], nidx2)

    iota = lax.iota(jnp.int32, L)

    def repack(i, _):
        j = i * L + iota
        r = j // N
        c = j - r * N
        nflat[pl.ds(i * L, L)] = plsc.load_gather(nidx2, [r, c])
        return 0
    lax.fori_loop(0, BPW * N // L, repack, 0)

    def fire(ch, b):
        ds = [pltpu.async_copy(w_hbm.at[widx.at[pl.ds(ch * CH, CH)]],
                               wrows.at[b], sem),
              pltpu.async_copy(v_hbm.at[vidx.at[pl.ds(ch * CH, CH)]],
                               vrows.at[b], sem)]
        for k in range(N):
            ds.append(pltpu.async_copy(
                v_hbm.at[nflat.at[pl.ds((ch * N + k) * CH, CH)]],
                nrows.at[b, pl.ds(k * CH, CH)], sem))
        return ds

    pending = fire(0, 0)
    for ch in range(NCH):
        b = ch % 2
        nxt = fire(ch + 1, 1 - b) if ch + 1 < NCH else []
        for d in pending:
            d.wait()
        pending = nxt
        wcur, vcur, ncur = wrows.at[b], vrows.at[b], nrows.at[b]

        for g in range(CH // L):
            row = g * L + iota
            rowx5 = row * N

            def d_body(d, accs, row=row, rowx5=rowx5, wcur=wcur, vcur=vcur,
                       ncur=ncur):
                dcol = jnp.full((L,), d, jnp.int32)
                vv = plsc.load_gather(vcur, [row, dcol])
                wv = plsc.load_gather(wcur, [row, dcol])
                out = [accs[0] + wv * vv]
                for n in range(N):
                    nv = plsc.load_gather(ncur, [rowx5 + n, dcol])
                    out.append(accs[1 + n] + nv * vv)
                return tuple(out)

            z = jnp.zeros((L,), jnp.float32)
            accs = lax.fori_loop(0, D, d_body, (z,) * (1 + N))
            off = ch * CH + g * L
            outv[0, pl.ds(off, L)] = accs[0]
            for n in range(N):
                outv[1 + n, pl.ds(off, L)] = -accs[1 + n]

    for j in range(1 + N):
        pltpu.sync_copy(outv.at[j], out_hbm.at[pl.ds(j * B + obase, BPW)])


def _sc_dots(pos_w, pos_v, neg_v, w_emb, v_emb):
    mesh = plsc.VectorSubcoreMesh(core_axis_name="c", subcore_axis_name="s",
                                  num_cores=NC, num_subcores=NS)
    f = pl.kernel(
        _sc_body,
        out_type=jax.ShapeDtypeStruct(((1 + N) * B,), jnp.float32),
        mesh=mesh,
        compiler_params=pltpu.CompilerParams(needs_layout_passes=False,
                                             use_tc_tiling_on_sc=False),
        scratch_types=[
            pltpu.VMEM((BPW,), jnp.int32),
            pltpu.VMEM((BPW,), jnp.int32),
            pltpu.VMEM((BPW, N), jnp.int32),
            pltpu.VMEM((BPW * N,), jnp.int32),
            pltpu.VMEM((2, CH, D), jnp.float32),
            pltpu.VMEM((2, CH, D), jnp.float32),
            pltpu.VMEM((2, N * CH, D), jnp.float32),
            pltpu.VMEM((1 + N, BPW), jnp.float32),
            pltpu.SemaphoreType.DMA,
        ],
    )
    return f(pos_w, pos_v, neg_v, w_emb, v_emb)


def _tc_body(x_ref, o_ref):
    x = x_ref[...]
    y = jnp.minimum(x, 0.0) - jnp.log1p(jnp.exp(-jnp.abs(x)))
    o_ref[...] = -jnp.sum(y, keepdims=True)


def kernel(pos_w, pos_v, neg_v, w_emb, v_emb):
    raw = _sc_dots(pos_w.astype(jnp.int32), pos_v.astype(jnp.int32),
                   neg_v.astype(jnp.int32), w_emb, v_emb)
    loss = pl.pallas_call(
        _tc_body,
        out_shape=jax.ShapeDtypeStruct((1, 1), jnp.float32),
    )(raw.reshape((1 + N) * B // 128, 128))
    return loss[0, 0]
